# Initial kernel scaffold; baseline (speedup 1.0000x reference)
import jax
import jax.numpy as jnp
from jax.experimental import pallas as pl


def kernel(h, i, j, rbf, W1, b1, W2, b2):
    raise NotImplementedError("write your pallas kernel here")


# speedup vs baseline: 2.8637x; 2.8637x over previous
import functools

import jax
import jax.numpy as jnp
from jax import lax
from jax.experimental import pallas as pl
from jax.experimental.pallas import tpu as pltpu
from jax.experimental.pallas import tpu_sc as plsc

N_NODES = 10000
N_EDGES = 320000
HID = 128
NRBF = 16

NC, NS, LANES = 2, 16, 16
NW = NC * NS
EPG = 128
NG = N_EDGES // EPG
GITERS = -(-NG // NW)
NPAD = 10240
RPW = NPAD // NS

_mesh = plsc.VectorSubcoreMesh(core_axis_name="c", subcore_axis_name="s")


@functools.partial(
    pl.kernel,
    out_type=jax.ShapeDtypeStruct((N_EDGES, HID), jnp.float32),
    mesh=_mesh,
    scratch_types=[
        pltpu.VMEM((EPG,), jnp.int32),
        pltpu.VMEM((EPG, HID), jnp.float32),
        pltpu.SemaphoreType.DMA,
    ],
)
def _sc_gather(g_hbm, j_hbm, out_hbm, idx_v, rows_v, sem):
    wid = lax.axis_index("s") * NC + lax.axis_index("c")

    def step(it, carry):
        grp = it * NW + wid

        @pl.when(grp < NG)
        def _():
            base = grp * EPG
            pltpu.sync_copy(j_hbm.at[pl.ds(base, EPG)], idx_v)
            pltpu.async_copy(g_hbm.at[idx_v], rows_v, sem).wait()
            pltpu.sync_copy(rows_v, out_hbm.at[pl.ds(base, EPG)])

        return carry

    lax.fori_loop(0, GITERS, step, 0)


@functools.partial(
    pl.kernel,
    out_type=jax.ShapeDtypeStruct((NC, NPAD, HID), jnp.float32),
    mesh=_mesh,
    scratch_types=[
        pltpu.VMEM((EPG,), jnp.int32),
        pltpu.VMEM((EPG, HID), jnp.float32),
        pltpu.VMEM_SHARED((NPAD, HID), jnp.float32),
    ],
)
def _sc_scatter(i_hbm, a_hbm, zA_hbm, A_out, idx_v, rows_v, A_sh):
    cid = lax.axis_index("c")
    sid = lax.axis_index("s")
    wid = sid * NC + cid
    r0 = sid * RPW
    pltpu.sync_copy(zA_hbm, A_sh.at[pl.ds(r0, RPW)])
    plsc.subcore_barrier()

    def step(it, carry):
        grp = it * NW + wid

        @pl.when(grp < NG)
        def _():
            base = grp * EPG
            pltpu.sync_copy(i_hbm.at[pl.ds(base, EPG)], idx_v)
            pltpu.sync_copy(a_hbm.at[pl.ds(base, EPG)], rows_v)
            pltpu.sync_copy(rows_v, A_sh.at[idx_v], add=True)

        return carry

    lax.fori_loop(0, GITERS, step, 0)
    plsc.subcore_barrier()
    pltpu.sync_copy(A_sh.at[pl.ds(r0, RPW)], A_out.at[cid, pl.ds(r0, RPW)])


@functools.partial(
    pl.kernel,
    out_type=jax.ShapeDtypeStruct((NC, NPAD, HID), jnp.float32),
    mesh=_mesh,
    scratch_types=[
        pltpu.VMEM((EPG,), jnp.int32),
        pltpu.VMEM((EPG, HID), jnp.float32),
        pltpu.VMEM_SHARED((NPAD, HID), jnp.float32),
    ],
)
def _sc_deg(i_hbm, zA_hbm, ones_hbm, deg_out, idx_v, ones_v, deg_sh):
    cid = lax.axis_index("c")
    sid = lax.axis_index("s")
    wid = sid * NC + cid
    r0 = sid * RPW
    pltpu.sync_copy(zA_hbm, deg_sh.at[pl.ds(r0, RPW)])
    pltpu.sync_copy(ones_hbm, ones_v)
    plsc.subcore_barrier()

    def step(it, carry):
        grp = it * NW + wid

        @pl.when(grp < NG)
        def _():
            base = grp * EPG
            pltpu.sync_copy(i_hbm.at[pl.ds(base, EPG)], idx_v)
            pltpu.sync_copy(ones_v, deg_sh.at[idx_v], add=True)

        return carry

    lax.fori_loop(0, GITERS, step, 0)
    plsc.subcore_barrier()
    pltpu.sync_copy(deg_sh.at[pl.ds(r0, RPW)], deg_out.at[cid, pl.ds(r0, RPW)])


def _g_body(h_ref, w_ref, o_ref):
    o_ref[...] = h_ref[...] @ w_ref[...]


def _edge_body(gj_ref, rbf_ref, w_ref, b_ref, o_ref):
    pre = gj_ref[...] + rbf_ref[...] @ w_ref[...] + b_ref[...]
    o_ref[...] = pre * (1.0 / (1.0 + jnp.exp(-pre)))


def _out_body(h_ref, A_ref, deg_ref, w2_ref, b2_ref, o_ref):
    A = A_ref[0] + A_ref[1]
    deg = deg_ref[0, :, 0:1] + deg_ref[1, :, 0:1]
    o_ref[...] = h_ref[...] + A @ w2_ref[...] + deg * b2_ref[...]


def _tc_g(h, w):
    B = 2000
    return pl.pallas_call(
        _g_body,
        grid=(N_NODES // B,),
        in_specs=[
            pl.BlockSpec((B, HID), lambda n: (n, 0)),
            pl.BlockSpec((HID, HID), lambda n: (0, 0)),
        ],
        out_specs=pl.BlockSpec((B, HID), lambda n: (n, 0)),
        out_shape=jax.ShapeDtypeStruct((N_NODES, HID), jnp.float32),
    )(h, w)


def _tc_edge(gj, rbf, w, b):
    B = 1600
    return pl.pallas_call(
        _edge_body,
        grid=(N_EDGES // B,),
        in_specs=[
            pl.BlockSpec((B, HID), lambda n: (n, 0)),
            pl.BlockSpec((B, NRBF), lambda n: (n, 0)),
            pl.BlockSpec((NRBF, HID), lambda n: (0, 0)),
            pl.BlockSpec((1, HID), lambda n: (0, 0)),
        ],
        out_specs=pl.BlockSpec((B, HID), lambda n: (n, 0)),
        out_shape=jax.ShapeDtypeStruct((N_EDGES, HID), jnp.float32),
    )(gj, rbf, w, b)


def _tc_out(h, A, deg, w2, b2):
    B = 2000
    return pl.pallas_call(
        _out_body,
        grid=(N_NODES // B,),
        in_specs=[
            pl.BlockSpec((B, HID), lambda n: (n, 0)),
            pl.BlockSpec((NC, B, HID), lambda n: (0, n, 0)),
            pl.BlockSpec((NC, B, HID), lambda n: (0, n, 0)),
            pl.BlockSpec((HID, HID), lambda n: (0, 0)),
            pl.BlockSpec((1, HID), lambda n: (0, 0)),
        ],
        out_specs=pl.BlockSpec((B, HID), lambda n: (n, 0)),
        out_shape=jax.ShapeDtypeStruct((N_NODES, HID), jnp.float32),
    )(h, A, deg, w2, b2)


def kernel(h, i, j, rbf, W1, b1, W2, b2):
    i32 = i.astype(jnp.int32)
    g = _tc_g(h, W1[:HID])
    gj = _sc_gather(g, j.astype(jnp.int32))
    a = _tc_edge(gj, rbf, W1[HID:], b1.reshape(1, HID))
    zA = jnp.zeros((RPW, HID), jnp.float32)
    ones = jnp.ones((EPG, HID), jnp.float32)
    A = _sc_scatter(i32, a, zA)
    deg = _sc_deg(i32, zA, ones)
    return _tc_out(h, A, deg, W2, b2.reshape(1, HID))
